# baseline (speedup 1.0000x reference)
import jax
import jax.numpy as jnp
from jax.experimental import pallas as pl
from jax.experimental.pallas import tpu as pltpu

_BB = 32


def _mm_kernel(x_ref, w_ref, b_ref, o_ref):
    x = x_ref[...].astype(jnp.bfloat16)
    o_ref[...] = (
        jax.lax.dot(x, w_ref[...], preferred_element_type=jnp.float32)
        + b_ref[...]
    )


def kernel(X, W, b):
    B, K = X.shape
    _, N = W.shape
    w16 = W.astype(jnp.bfloat16)
    b2 = b.reshape(1, N)
    return pl.pallas_call(
        _mm_kernel,
        grid=(B // _BB,),
        in_specs=[
            pl.BlockSpec((_BB, K), lambda i: (i, 0)),
            pl.BlockSpec((K, N), lambda i: (0, 0)),
            pl.BlockSpec((1, N), lambda i: (0, 0)),
        ],
        out_specs=pl.BlockSpec((_BB, N), lambda i: (i, 0)),
        out_shape=jax.ShapeDtypeStruct((B, N), jnp.float32),
        compiler_params=pltpu.CompilerParams(
            dimension_semantics=("parallel",),
        ),
    )(X, w16, b2)
